# baseline (speedup 1.0000x reference)
import functools

import jax
import jax.numpy as jnp
import numpy as np
from jax import lax
from jax.experimental import pallas as pl
from jax.experimental.pallas import tpu as pltpu
from jax.experimental.pallas import tpu_sc as plsc

_SR = 0.5

_NC = 2
_NS = 16
_L = 16
_NW = _NC * _NS

_SIGN = np.int32(-2**31)


def _f32_to_key(v):
    y = plsc.bitcast(v, jnp.int32)
    return jnp.where(y < 0, ~y, y ^ _SIGN)


def _scan_hist(hist_ref, base0, base1, nbins, krem):
    nchunks = nbins // _L
    iota = lax.iota(jnp.int32, _L)

    acc = jnp.int32(0)
    kr = krem
    found = jnp.bool_(False)
    bsel = jnp.int32(0)
    for j in range(nchunks - 1, -1, -1):
        bins = (hist_ref[pl.ds(base0 + j * _L, _L)]
                + hist_ref[pl.ds(base1 + j * _L, _L)])
        c = plsc.cumsum(bins)
        total = jnp.max(c)
        excl = c - bins
        rhs = acc + total - krem
        cond = excl <= rhs
        p = jnp.max(plsc.all_reduce_population_count(cond))
        newly = jnp.logical_and(jnp.logical_not(found), p > 0)
        local = p - 1
        c_at = jnp.sum(jnp.where(iota == local, c, 0))
        count_above = acc + total - c_at
        bsel = jnp.where(newly, j * _L + local, bsel)
        kr = jnp.where(newly, krem - count_above, kr)
        found = jnp.logical_or(found, p > 0)
        acc = acc + total
    return bsel, kr


def _zero_hist(hist_ref, nbins):
    zeros = jnp.zeros((_L,), jnp.int32)
    for i in range(nbins // _L):
        hist_ref[pl.ds(i * _L, _L)] = zeros


def _sc_thresholds(x, rows, cols, k):
    nvec = cols // _L
    nbins = 256
    unroll = 8
    nhist = 8
    mesh = plsc.VectorSubcoreMesh(core_axis_name="c", subcore_axis_name="s")
    rows_per_w = rows // _NW

    @functools.partial(
        pl.kernel,
        mesh=mesh,
        out_type=jax.ShapeDtypeStruct((_NW, _L), jnp.int32),
        scratch_types=[
            pltpu.VMEM((cols,), jnp.float32),
            pltpu.VMEM((cols,), jnp.float32),
            pltpu.VMEM((cols,), jnp.int32),
            pltpu.VMEM((nbins * nhist,), jnp.int32),
            pltpu.VMEM((_L,), jnp.int32),
            pltpu.SemaphoreType.DMA,
            pltpu.SemaphoreType.DMA,
        ],
        compiler_params=pltpu.CompilerParams(needs_layout_passes=False),
    )
    def thresh_kernel(x_hbm, out_hbm, data_v0, data_v1, key_v, hist_v,
                      thr_v, sem0, sem1):
        c = lax.axis_index("c")
        s = lax.axis_index("s")
        wid = s * _NC + c
        iota = lax.iota(jnp.int32, _L)
        ones = jnp.ones((_L,), jnp.int32)

        thr_v[...] = jnp.zeros((_L,), jnp.int32)

        bufs = [(data_v0, sem0), (data_v1, sem1)]
        copies = []
        for r in range(rows_per_w):
            data_v, sem = bufs[r % 2]
            copies.append(
                pltpu.async_copy(x_hbm.at[wid * rows_per_w + r], data_v, sem))

        for r in range(rows_per_w):
            data_v, _ = bufs[r % 2]
            copies[r].wait()

            _zero_hist(hist_v, nbins * nhist)

            @plsc.parallel_loop(0, nvec, step=2, unroll=unroll)
            def pass1(j):
                for q in range(2):
                    v = data_v[pl.ds((j + q) * _L, _L)]
                    kv = _f32_to_key(v)
                    key_v[pl.ds((j + q) * _L, _L)] = kv
                    idx = lax.shift_right_logical(kv, 24) | (q * nbins)
                    plsc.addupdate_scatter(hist_v, [idx], ones)

            b1, krem = _scan_hist(hist_v, 0, nbins, nbins, jnp.int32(k))

            def refine(p, pref, shift, krem):
                @plsc.parallel_loop(0, nvec, step=2, unroll=unroll)
                def body(j):
                    for q in range(2):
                        kv = key_v[pl.ds((j + q) * _L, _L)]
                        m = lax.shift_right_logical(kv, shift + 8) == pref
                        idx = (lax.shift_right_logical(kv, shift) & 0xFF) | (
                            (2 * p + q) * nbins)
                        plsc.addupdate_scatter(hist_v, [idx], ones, mask=m)

                b, krem = _scan_hist(hist_v, 2 * p * nbins,
                                     (2 * p + 1) * nbins, nbins, krem)
                return (pref << 8) | b, krem

            pref, krem = refine(1, b1, 16, krem)
            pref, krem = refine(2, pref, 8, krem)
            tkey, _ = refine(3, pref, 0, krem)

            thr_v[...] = jnp.where(iota == r, tkey, thr_v[...])

        pltpu.sync_copy(thr_v, out_hbm.at[wid])

    return thresh_kernel(x)


_B16 = np.uint16(0x8000)


def _bias16(a_u16):
    return lax.bitcast_convert_type(a_u16 ^ _B16, jnp.int16)


def _count_ge_m1(a_s, cand_s):
    n = a_s.shape[1]
    c1 = n // 16
    ind = (a_s >= cand_s).astype(jnp.int16)
    acc = ind[:, :c1]
    for j in range(1, 16):
        acc = acc + ind[:, j * c1:(j + 1) * c1]
    c2 = c1 // 16
    acc2 = acc[:, :c2]
    for j in range(1, 16):
        acc2 = acc2 + acc[:, j * c2:(j + 1) * c2]
    cnt = jnp.sum(acc2.astype(jnp.int32), axis=1, keepdims=True)
    return (cnt - 1).astype(jnp.int16)


def _descend_u16(a_s, km1):
    blk = a_s.shape[0]
    u = jnp.zeros((blk, 1), dtype=jnp.uint16)
    for b in range(15, -1, -1):
        cand = u | jnp.uint16(1 << b)
        cntm1 = _count_ge_m1(a_s, _bias16(cand))
        u = jnp.where(cntm1 >= km1, cand, u)
    return u


def _tc_thresh_block(x_ref, t_ref, *, k):
    x = x_ref[...]
    y = lax.bitcast_convert_type(x, jnp.uint32)
    sign = jnp.uint32(0x80000000)
    ukey = jnp.where(y >= sign, ~y, y ^ sign)
    hi = lax.shift_right_logical(ukey, jnp.uint32(16)).astype(jnp.uint16)
    lo = (ukey & jnp.uint32(0xFFFF)).astype(jnp.uint16)

    blk = x.shape[0]
    km1 = jnp.full((blk, 1), k - 1, dtype=jnp.int16)
    hi_s = _bias16(hi)

    t_hi = _descend_u16(hi_s, km1)

    sat = t_hi == jnp.uint16(0xFFFF)
    cgm1 = _count_ge_m1(hi_s, _bias16(t_hi + jnp.uint16(1)))
    c_gtm1 = jnp.where(sat, jnp.int16(-1), cgm1)
    k2m1 = km1 - c_gtm1 - jnp.int16(1)

    lo_m = jnp.where(hi == t_hi, lo, jnp.uint16(0))
    t_lo = _descend_u16(_bias16(lo_m), k2m1)

    t32 = (t_hi.astype(jnp.uint32) << 16) | t_lo.astype(jnp.uint32)
    tbits = jnp.where(t32 >= sign, t32 ^ sign, ~t32)
    t_ref[...] = lax.bitcast_convert_type(tbits, jnp.float32)


def _tc_thresholds(x, k, row_start, nrows, blk=16):
    cols = x.shape[1]
    off = row_start // blk
    return pl.pallas_call(
        functools.partial(_tc_thresh_block, k=k),
        grid=(nrows // blk,),
        in_specs=[pl.BlockSpec((blk, cols), lambda i: (i + off, 0))],
        out_specs=pl.BlockSpec((blk, 1), lambda i: (i, 0)),
        out_shape=jax.ShapeDtypeStruct((nrows, 1), jnp.float32),
    )(x)


def _mask_block(x_ref, t_ref, o_ref):
    x = x_ref[...]
    t = t_ref[...]
    o_ref[...] = jnp.where(x >= t, x, jnp.float32(0.0))


_SC_ROWS = 64


@jax.jit
def kernel(x):
    rows, cols = x.shape
    k = int(_SR * cols)

    thr_tc = _tc_thresholds(x, k, _SC_ROWS, rows - _SC_ROWS)

    tkeys = _sc_thresholds(x, _SC_ROWS, cols, k)
    rows_per_w = _SC_ROWS // _NW
    tkeys = tkeys[:, :rows_per_w].reshape(_SC_ROWS, 1)
    tbits = jnp.where(tkeys < 0, tkeys ^ _SIGN, ~tkeys)
    thr_sc = lax.bitcast_convert_type(tbits, jnp.float32)
    thr = jnp.concatenate([thr_sc, thr_tc], axis=0)

    blk = 16
    grid = (rows // blk,)
    return pl.pallas_call(
        _mask_block,
        grid=grid,
        in_specs=[
            pl.BlockSpec((blk, cols), lambda i: (i, 0)),
            pl.BlockSpec((blk, 1), lambda i: (i, 0)),
        ],
        out_specs=pl.BlockSpec((blk, cols), lambda i: (i, 0)),
        out_shape=jax.ShapeDtypeStruct((rows, cols), x.dtype),
    )(x, thr)


# speedup vs baseline: 1.0406x; 1.0406x over previous
import functools

import jax
import jax.numpy as jnp
import numpy as np
from jax import lax
from jax.experimental import pallas as pl
from jax.experimental.pallas import tpu as pltpu
from jax.experimental.pallas import tpu_sc as plsc

_SR = 0.5

_NC = 2
_NS = 16
_L = 16
_NW = _NC * _NS

_SIGN = np.int32(-2**31)


def _f32_to_key(v):
    y = plsc.bitcast(v, jnp.int32)
    return jnp.where(y < 0, ~y, y ^ _SIGN)


def _scan_hist(hist_ref, base0, base1, nbins, krem):
    nchunks = nbins // _L
    iota = lax.iota(jnp.int32, _L)

    acc = jnp.int32(0)
    kr = krem
    found = jnp.bool_(False)
    bsel = jnp.int32(0)
    for j in range(nchunks - 1, -1, -1):
        bins = (hist_ref[pl.ds(base0 + j * _L, _L)]
                + hist_ref[pl.ds(base1 + j * _L, _L)])
        c = plsc.cumsum(bins)
        total = jnp.max(c)
        excl = c - bins
        rhs = acc + total - krem
        cond = excl <= rhs
        p = jnp.max(plsc.all_reduce_population_count(cond))
        newly = jnp.logical_and(jnp.logical_not(found), p > 0)
        local = p - 1
        c_at = jnp.sum(jnp.where(iota == local, c, 0))
        count_above = acc + total - c_at
        bsel = jnp.where(newly, j * _L + local, bsel)
        kr = jnp.where(newly, krem - count_above, kr)
        found = jnp.logical_or(found, p > 0)
        acc = acc + total
    return bsel, kr


def _zero_hist(hist_ref, nbins):
    zeros = jnp.zeros((_L,), jnp.int32)
    for i in range(nbins // _L):
        hist_ref[pl.ds(i * _L, _L)] = zeros


def _sc_thresholds(x, rows, cols, k):
    nvec = cols // _L
    nbins = 256
    unroll = 8
    nhist = 8
    mesh = plsc.VectorSubcoreMesh(core_axis_name="c", subcore_axis_name="s")
    rows_per_w = rows // _NW

    @functools.partial(
        pl.kernel,
        mesh=mesh,
        out_type=jax.ShapeDtypeStruct((_NW, _L), jnp.int32),
        scratch_types=[
            pltpu.VMEM((cols,), jnp.float32),
            pltpu.VMEM((cols,), jnp.int32),
            pltpu.VMEM((nbins * nhist,), jnp.int32),
            pltpu.VMEM((_L,), jnp.int32),
        ],
        compiler_params=pltpu.CompilerParams(needs_layout_passes=False),
    )
    def thresh_kernel(x_hbm, out_hbm, data_v, key_v, hist_v, thr_v):
        c = lax.axis_index("c")
        s = lax.axis_index("s")
        wid = s * _NC + c
        iota = lax.iota(jnp.int32, _L)
        ones = jnp.ones((_L,), jnp.int32)

        thr_v[...] = jnp.zeros((_L,), jnp.int32)

        def row_body(r, _):
            row = wid * rows_per_w + r
            pltpu.sync_copy(x_hbm.at[row], data_v)

            _zero_hist(hist_v, nbins * nhist)

            @plsc.parallel_loop(0, nvec, step=2, unroll=unroll)
            def pass1(j):
                for q in range(2):
                    v = data_v[pl.ds((j + q) * _L, _L)]
                    kv = _f32_to_key(v)
                    key_v[pl.ds((j + q) * _L, _L)] = kv
                    idx = lax.shift_right_logical(kv, 24) | (q * nbins)
                    plsc.addupdate_scatter(hist_v, [idx], ones)

            b1, krem = _scan_hist(hist_v, 0, nbins, nbins, jnp.int32(k))

            def refine(p, pref, shift, krem):
                @plsc.parallel_loop(0, nvec, step=2, unroll=unroll)
                def body(j):
                    for q in range(2):
                        kv = key_v[pl.ds((j + q) * _L, _L)]
                        m = lax.shift_right_logical(kv, shift + 8) == pref
                        idx = (lax.shift_right_logical(kv, shift) & 0xFF) | (
                            (2 * p + q) * nbins)
                        plsc.addupdate_scatter(hist_v, [idx], ones, mask=m)

                b, krem = _scan_hist(hist_v, 2 * p * nbins,
                                     (2 * p + 1) * nbins, nbins, krem)
                return (pref << 8) | b, krem

            pref, krem = refine(1, b1, 16, krem)
            pref, krem = refine(2, pref, 8, krem)
            tkey, _ = refine(3, pref, 0, krem)

            thr_v[...] = jnp.where(iota == r, tkey, thr_v[...])
            return 0

        lax.fori_loop(0, rows_per_w, row_body, 0)
        pltpu.sync_copy(thr_v, out_hbm.at[wid])

    return thresh_kernel(x)


_B16 = np.uint16(0x8000)


def _bias16(a_u16):
    return lax.bitcast_convert_type(a_u16 ^ _B16, jnp.int16)


def _count_ge_m1(a_s, cand_s):
    n = a_s.shape[1]
    c1 = n // 16
    ind = (a_s >= cand_s).astype(jnp.int16)
    acc = ind[:, :c1]
    for j in range(1, 16):
        acc = acc + ind[:, j * c1:(j + 1) * c1]
    c2 = c1 // 16
    acc2 = acc[:, :c2]
    for j in range(1, 16):
        acc2 = acc2 + acc[:, j * c2:(j + 1) * c2]
    cnt = jnp.sum(acc2.astype(jnp.int32), axis=1, keepdims=True)
    return (cnt - 1).astype(jnp.int16)


def _descend_u16(a_s, km1):
    blk = a_s.shape[0]
    u = jnp.zeros((blk, 1), dtype=jnp.uint16)
    for b in range(15, -1, -1):
        cand = u | jnp.uint16(1 << b)
        cntm1 = _count_ge_m1(a_s, _bias16(cand))
        u = jnp.where(cntm1 >= km1, cand, u)
    return u


def _tc_thresh_block(x_ref, t_ref, *, k):
    x = x_ref[...]
    y = lax.bitcast_convert_type(x, jnp.uint32)
    sign = jnp.uint32(0x80000000)
    ukey = jnp.where(y >= sign, ~y, y ^ sign)
    hi = lax.shift_right_logical(ukey, jnp.uint32(16)).astype(jnp.uint16)
    lo = (ukey & jnp.uint32(0xFFFF)).astype(jnp.uint16)

    blk = x.shape[0]
    km1 = jnp.full((blk, 1), k - 1, dtype=jnp.int16)
    hi_s = _bias16(hi)

    t_hi = _descend_u16(hi_s, km1)

    sat = t_hi == jnp.uint16(0xFFFF)
    cgm1 = _count_ge_m1(hi_s, _bias16(t_hi + jnp.uint16(1)))
    c_gtm1 = jnp.where(sat, jnp.int16(-1), cgm1)
    k2m1 = km1 - c_gtm1 - jnp.int16(1)

    lo_m = jnp.where(hi == t_hi, lo, jnp.uint16(0))
    t_lo = _descend_u16(_bias16(lo_m), k2m1)

    t32 = (t_hi.astype(jnp.uint32) << 16) | t_lo.astype(jnp.uint32)
    tbits = jnp.where(t32 >= sign, t32 ^ sign, ~t32)
    t_ref[...] = lax.bitcast_convert_type(tbits, jnp.float32)


def _tc_thresholds(x, k, row_start, nrows, blk=16):
    cols = x.shape[1]
    off = row_start // blk
    return pl.pallas_call(
        functools.partial(_tc_thresh_block, k=k),
        grid=(nrows // blk,),
        in_specs=[pl.BlockSpec((blk, cols), lambda i: (i + off, 0))],
        out_specs=pl.BlockSpec((blk, 1), lambda i: (i, 0)),
        out_shape=jax.ShapeDtypeStruct((nrows, 1), jnp.float32),
    )(x)


def _mask_block(x_ref, t_ref, o_ref):
    x = x_ref[...]
    t = t_ref[...]
    o_ref[...] = jnp.where(x >= t, x, jnp.float32(0.0))


_SC_ROWS = 64


@jax.jit
def kernel(x):
    rows, cols = x.shape
    k = int(_SR * cols)

    thr_tc = _tc_thresholds(x, k, _SC_ROWS, rows - _SC_ROWS)

    tkeys = _sc_thresholds(x, _SC_ROWS, cols, k)
    rows_per_w = _SC_ROWS // _NW
    tkeys = tkeys[:, :rows_per_w].reshape(_SC_ROWS, 1)
    tbits = jnp.where(tkeys < 0, tkeys ^ _SIGN, ~tkeys)
    thr_sc = lax.bitcast_convert_type(tbits, jnp.float32)
    thr = jnp.concatenate([thr_sc, thr_tc], axis=0)

    blk = 16
    grid = (rows // blk,)
    return pl.pallas_call(
        _mask_block,
        grid=grid,
        in_specs=[
            pl.BlockSpec((blk, cols), lambda i: (i, 0)),
            pl.BlockSpec((blk, 1), lambda i: (i, 0)),
        ],
        out_specs=pl.BlockSpec((blk, cols), lambda i: (i, 0)),
        out_shape=jax.ShapeDtypeStruct((rows, cols), x.dtype),
    )(x, thr)


# speedup vs baseline: 1.0424x; 1.0017x over previous
import functools

import jax
import jax.numpy as jnp
import numpy as np
from jax import lax
from jax.experimental import pallas as pl
from jax.experimental.pallas import tpu as pltpu
from jax.experimental.pallas import tpu_sc as plsc

_SR = 0.5

_NC = 2
_NS = 16
_L = 16
_NW = _NC * _NS

_SIGN = np.int32(-2**31)


def _f32_to_key(v):
    y = plsc.bitcast(v, jnp.int32)
    return jnp.where(y < 0, ~y, y ^ _SIGN)


def _scan_hist(hist_ref, base0, base1, nbins, krem):
    nchunks = nbins // _L
    iota = lax.iota(jnp.int32, _L)

    acc = jnp.int32(0)
    kr = krem
    found = jnp.bool_(False)
    bsel = jnp.int32(0)
    for j in range(nchunks - 1, -1, -1):
        bins = (hist_ref[pl.ds(base0 + j * _L, _L)]
                + hist_ref[pl.ds(base1 + j * _L, _L)])
        c = plsc.cumsum(bins)
        total = jnp.max(c)
        excl = c - bins
        rhs = acc + total - krem
        cond = excl <= rhs
        p = jnp.max(plsc.all_reduce_population_count(cond))
        newly = jnp.logical_and(jnp.logical_not(found), p > 0)
        local = p - 1
        c_at = jnp.sum(jnp.where(iota == local, c, 0))
        count_above = acc + total - c_at
        bsel = jnp.where(newly, j * _L + local, bsel)
        kr = jnp.where(newly, krem - count_above, kr)
        found = jnp.logical_or(found, p > 0)
        acc = acc + total
    return bsel, kr


def _zero_hist(hist_ref, nbins):
    zeros = jnp.zeros((_L,), jnp.int32)
    for i in range(nbins // _L):
        hist_ref[pl.ds(i * _L, _L)] = zeros


def _sc_thresholds(x, rows, cols, k):
    nvec = cols // _L
    nbins = 256
    unroll = 8
    nhist = 8
    mesh = plsc.VectorSubcoreMesh(core_axis_name="c", subcore_axis_name="s")
    rows_per_w = rows // _NW

    @functools.partial(
        pl.kernel,
        mesh=mesh,
        out_type=jax.ShapeDtypeStruct((_NW, _L), jnp.int32),
        scratch_types=[
            pltpu.VMEM((cols,), jnp.float32),
            pltpu.VMEM((cols,), jnp.int32),
            pltpu.VMEM((nbins * nhist,), jnp.int32),
            pltpu.VMEM((_L,), jnp.int32),
        ],
        compiler_params=pltpu.CompilerParams(needs_layout_passes=False),
    )
    def thresh_kernel(x_hbm, out_hbm, data_v, key_v, hist_v, thr_v):
        c = lax.axis_index("c")
        s = lax.axis_index("s")
        wid = s * _NC + c
        iota = lax.iota(jnp.int32, _L)
        ones = jnp.ones((_L,), jnp.int32)

        thr_v[...] = jnp.zeros((_L,), jnp.int32)

        def row_body(r, _):
            row = wid * rows_per_w + r
            pltpu.sync_copy(x_hbm.at[row], data_v)

            _zero_hist(hist_v, nbins * nhist)

            @plsc.parallel_loop(0, nvec, step=2, unroll=unroll)
            def pass1(j):
                for q in range(2):
                    v = data_v[pl.ds((j + q) * _L, _L)]
                    kv = _f32_to_key(v)
                    key_v[pl.ds((j + q) * _L, _L)] = kv
                    idx = lax.shift_right_logical(kv, 24) | (q * nbins)
                    plsc.addupdate_scatter(hist_v, [idx], ones)

            b1, krem = _scan_hist(hist_v, 0, nbins, nbins, jnp.int32(k))

            def refine(p, pref, shift, krem):
                @plsc.parallel_loop(0, nvec, step=2, unroll=unroll)
                def body(j):
                    for q in range(2):
                        kv = key_v[pl.ds((j + q) * _L, _L)]
                        m = lax.shift_right_logical(kv, shift + 8) == pref
                        idx = (lax.shift_right_logical(kv, shift) & 0xFF) | (
                            (2 * p + q) * nbins)
                        plsc.addupdate_scatter(hist_v, [idx], ones, mask=m)

                b, krem = _scan_hist(hist_v, 2 * p * nbins,
                                     (2 * p + 1) * nbins, nbins, krem)
                return (pref << 8) | b, krem

            pref, krem = refine(1, b1, 16, krem)
            pref, krem = refine(2, pref, 8, krem)
            tkey, _ = refine(3, pref, 0, krem)

            thr_v[...] = jnp.where(iota == r, tkey, thr_v[...])
            return 0

        lax.fori_loop(0, rows_per_w, row_body, 0)
        pltpu.sync_copy(thr_v, out_hbm.at[wid])

    return thresh_kernel(x)


_B16 = np.uint16(0x8000)


def _bias16(a_u16):
    return lax.bitcast_convert_type(a_u16 ^ _B16, jnp.int16)


def _count_ge_m1(a_s, cand_s):
    n = a_s.shape[1]
    c1 = n // 16
    ind = (a_s >= cand_s).astype(jnp.int16)
    acc = ind[:, :c1]
    for j in range(1, 16):
        acc = acc + ind[:, j * c1:(j + 1) * c1]
    c2 = c1 // 16
    acc2 = acc[:, :c2]
    for j in range(1, 16):
        acc2 = acc2 + acc[:, j * c2:(j + 1) * c2]
    cnt = jnp.sum(acc2.astype(jnp.int32), axis=1, keepdims=True)
    return (cnt - 1).astype(jnp.int16)


def _descend_u16(a_s, km1):
    blk = a_s.shape[0]
    u = jnp.zeros((blk, 1), dtype=jnp.uint16)
    for b in range(15, -1, -1):
        cand = u | jnp.uint16(1 << b)
        cntm1 = _count_ge_m1(a_s, _bias16(cand))
        u = jnp.where(cntm1 >= km1, cand, u)
    return u


def _tc_thresh_block(x_ref, t_ref, *, k):
    x = x_ref[...]
    y = lax.bitcast_convert_type(x, jnp.uint32)
    sign = jnp.uint32(0x80000000)
    ukey = jnp.where(y >= sign, ~y, y ^ sign)
    hi = lax.shift_right_logical(ukey, jnp.uint32(16)).astype(jnp.uint16)
    lo = (ukey & jnp.uint32(0xFFFF)).astype(jnp.uint16)

    blk = x.shape[0]
    km1 = jnp.full((blk, 1), k - 1, dtype=jnp.int16)
    hi_s = _bias16(hi)

    t_hi = _descend_u16(hi_s, km1)

    sat = t_hi == jnp.uint16(0xFFFF)
    cgm1 = _count_ge_m1(hi_s, _bias16(t_hi + jnp.uint16(1)))
    c_gtm1 = jnp.where(sat, jnp.int16(-1), cgm1)
    k2m1 = km1 - c_gtm1 - jnp.int16(1)

    lo_m = jnp.where(hi == t_hi, lo, jnp.uint16(0))
    t_lo = _descend_u16(_bias16(lo_m), k2m1)

    t32 = (t_hi.astype(jnp.uint32) << 16) | t_lo.astype(jnp.uint32)
    tbits = jnp.where(t32 >= sign, t32 ^ sign, ~t32)
    t_ref[...] = lax.bitcast_convert_type(tbits, jnp.float32)


def _tc_thresholds(x, k, row_start, nrows, blk=32):
    cols = x.shape[1]
    off = row_start // blk
    return pl.pallas_call(
        functools.partial(_tc_thresh_block, k=k),
        grid=(nrows // blk,),
        in_specs=[pl.BlockSpec((blk, cols), lambda i: (i + off, 0))],
        out_specs=pl.BlockSpec((blk, 1), lambda i: (i, 0)),
        out_shape=jax.ShapeDtypeStruct((nrows, 1), jnp.float32),
    )(x)


def _mask_block(x_ref, t_ref, o_ref):
    x = x_ref[...]
    t = t_ref[...]
    o_ref[...] = jnp.where(x >= t, x, jnp.float32(0.0))


_SC_ROWS = 64


@jax.jit
def kernel(x):
    rows, cols = x.shape
    k = int(_SR * cols)

    thr_tc = _tc_thresholds(x, k, _SC_ROWS, rows - _SC_ROWS)

    tkeys = _sc_thresholds(x, _SC_ROWS, cols, k)
    rows_per_w = _SC_ROWS // _NW
    tkeys = tkeys[:, :rows_per_w].reshape(_SC_ROWS, 1)
    tbits = jnp.where(tkeys < 0, tkeys ^ _SIGN, ~tkeys)
    thr_sc = lax.bitcast_convert_type(tbits, jnp.float32)
    thr = jnp.concatenate([thr_sc, thr_tc], axis=0)

    blk = 16
    grid = (rows // blk,)
    return pl.pallas_call(
        _mask_block,
        grid=grid,
        in_specs=[
            pl.BlockSpec((blk, cols), lambda i: (i, 0)),
            pl.BlockSpec((blk, 1), lambda i: (i, 0)),
        ],
        out_specs=pl.BlockSpec((blk, cols), lambda i: (i, 0)),
        out_shape=jax.ShapeDtypeStruct((rows, cols), x.dtype),
    )(x, thr)
